# Initial kernel scaffold; baseline (speedup 1.0000x reference)
import jax
import jax.numpy as jnp
from jax.experimental import pallas as pl


def kernel(input):
    raise NotImplementedError("write your pallas kernel here")


# speedup vs baseline: 7.8215x; 7.8215x over previous
import functools

import jax
import jax.numpy as jnp
from jax import lax
from jax.experimental import pallas as pl
from jax.experimental.pallas import tpu as pltpu
from jax.experimental.pallas import tpu_sc as plsc

LANES = 16
NW = 32


def _vgather(x, idx):
    return lax.gather(
        x,
        idx[:, None],
        dimension_numbers=lax.GatherDimensionNumbers(
            offset_dims=(), collapsed_slice_dims=(0,), start_index_map=(0,)
        ),
        slice_sizes=(1,),
        mode=lax.GatherScatterMode.PROMISE_IN_BOUNDS,
    )


def _align_vreg(x, o16, base16, p1, p2, p4):
    bits = lax.bitcast_convert_type(x, jnp.int32)
    key = bits ^ (lax.shift_right_arithmetic(bits, 31) & jnp.int32(0x7FFFFFFF))
    key = (key & jnp.int32(-8)) | (7 - o16)
    key = jnp.maximum(key, _vgather(key, p1))
    key = jnp.maximum(key, _vgather(key, p2))
    key = jnp.maximum(key, _vgather(key, p4))
    d = 7 - (key & 7)
    idx = base16 + ((o16 + d) & 7)
    return _vgather(x, idx)


def kernel(input):
    N, C = input.shape
    total = N * C
    per_w = total // NW
    chunk = 16384
    n_chunks = per_w // chunk
    n_vregs = chunk // LANES

    mesh = plsc.VectorSubcoreMesh(core_axis_name="c", subcore_axis_name="s")

    @functools.partial(
        pl.kernel,
        mesh=mesh,
        out_type=jax.ShapeDtypeStruct((total,), jnp.float32),
        scratch_types=[
            pltpu.VMEM((chunk,), jnp.float32),
            pltpu.VMEM((chunk,), jnp.float32),
        ],
    )
    def run(x_hbm, out_hbm, ibuf, obuf):
        wid = lax.axis_index("s") * 2 + lax.axis_index("c")
        base = wid * per_w
        iota = lax.iota(jnp.int32, LANES)
        o16 = iota & 7
        base16 = iota & jnp.int32(-8)
        p1 = iota ^ 1
        p2 = iota ^ 2
        p4 = iota ^ 4

        def do_chunk(ci, _):
            off = base + ci * chunk
            pltpu.sync_copy(x_hbm.at[pl.ds(off, chunk)], ibuf)

            def do_vreg(v, _):
                x = ibuf[pl.ds(v * LANES, LANES)]
                obuf[pl.ds(v * LANES, LANES)] = _align_vreg(
                    x, o16, base16, p1, p2, p4
                )
                return 0

            lax.fori_loop(0, n_vregs, do_vreg, 0)
            pltpu.sync_copy(obuf, out_hbm.at[pl.ds(off, chunk)])
            return 0

        lax.fori_loop(0, n_chunks, do_chunk, 0)

    out = run(input.reshape(total))
    return out.reshape(N, C)


# speedup vs baseline: 9.0709x; 1.1597x over previous
import functools

import jax
import jax.numpy as jnp
from jax import lax
from jax.experimental import pallas as pl
from jax.experimental.pallas import tpu as pltpu
from jax.experimental.pallas import tpu_sc as plsc

LANES = 16
NW = 32


def _vgather(x, idx):
    return lax.gather(
        x,
        idx[:, None],
        dimension_numbers=lax.GatherDimensionNumbers(
            offset_dims=(), collapsed_slice_dims=(0,), start_index_map=(0,)
        ),
        slice_sizes=(1,),
        mode=lax.GatherScatterMode.PROMISE_IN_BOUNDS,
    )


def _align_vreg(x, o16, base16, p1, p2, p4):
    bits = lax.bitcast_convert_type(x, jnp.int32)
    key = bits ^ (lax.shift_right_arithmetic(bits, 31) & jnp.int32(0x7FFFFFFF))
    key = (key & jnp.int32(-8)) | (7 - o16)
    key = jnp.maximum(key, _vgather(key, p1))
    key = jnp.maximum(key, _vgather(key, p2))
    key = jnp.maximum(key, _vgather(key, p4))
    d = 7 - (key & 7)
    idx = base16 + ((o16 + d) & 7)
    return _vgather(x, idx)


def kernel(input):
    N, C = input.shape
    total = N * C
    per_w = total // NW
    chunk = 16384
    n_chunks = per_w // chunk
    n_vregs = chunk // LANES

    mesh = plsc.VectorSubcoreMesh(core_axis_name="c", subcore_axis_name="s")

    @functools.partial(
        pl.kernel,
        mesh=mesh,
        out_type=jax.ShapeDtypeStruct((total,), jnp.float32),
        scratch_types=[
            pltpu.VMEM((2, chunk), jnp.float32),
            pltpu.VMEM((2, chunk), jnp.float32),
            pltpu.SemaphoreType.DMA,
            pltpu.SemaphoreType.DMA,
            pltpu.SemaphoreType.DMA,
            pltpu.SemaphoreType.DMA,
        ],
    )
    def run(x_hbm, out_hbm, ibuf, obuf, si0, si1, so0, so1):
        wid = lax.axis_index("s") * 2 + lax.axis_index("c")
        base = wid * per_w
        iota = lax.iota(jnp.int32, LANES)
        o16 = iota & 7
        base16 = iota & jnp.int32(-8)
        p1 = iota ^ 1
        p2 = iota ^ 2
        p4 = iota ^ 4
        sem_in = (si0, si1)
        sem_out = (so0, so1)

        def in_slice(ci):
            return x_hbm.at[pl.ds(base + ci * chunk, chunk)]

        def out_slice(ci):
            return out_hbm.at[pl.ds(base + ci * chunk, chunk)]

        for b in range(2):
            pltpu.async_copy(in_slice(b), ibuf.at[b], sem_in[b])

        def outer(t, _):
            for b in range(2):
                ci = t * 2 + b
                ib = ibuf.at[b]
                ob = obuf.at[b]
                pltpu.make_async_copy(in_slice(ci), ib, sem_in[b]).wait()

                @pl.when(t > 0)
                def _():
                    pltpu.make_async_copy(ob, out_slice(ci - 2), sem_out[b]).wait()

                @plsc.parallel_loop(0, n_vregs, unroll=8)
                def body(v):
                    x = ib[pl.ds(v * LANES, LANES)]
                    ob[pl.ds(v * LANES, LANES)] = _align_vreg(
                        x, o16, base16, p1, p2, p4
                    )

                pltpu.async_copy(ob, out_slice(ci), sem_out[b])

                @pl.when(ci + 2 < n_chunks)
                def _():
                    pltpu.async_copy(in_slice(ci + 2), ibuf.at[b], sem_in[b])

            return 0

        lax.fori_loop(0, n_chunks // 2, outer, 0)

        for b in range(2):
            ci = n_chunks - 2 + b
            pltpu.make_async_copy(obuf.at[b], out_slice(ci), sem_out[b]).wait()

    out = run(input.reshape(total))
    return out.reshape(N, C)


# speedup vs baseline: 20.1997x; 2.2269x over previous
import functools

import jax
import jax.numpy as jnp
from jax import lax
from jax.experimental import pallas as pl
from jax.experimental.pallas import tpu as pltpu
from jax.experimental.pallas import tpu_sc as plsc

LANES = 16
NW = 32
CR = 8
NBUF = 3


def _vgather(x, idx):
    return lax.gather(
        x,
        idx[:, None],
        dimension_numbers=lax.GatherDimensionNumbers(
            offset_dims=(), collapsed_slice_dims=(0,), start_index_map=(0,)
        ),
        slice_sizes=(1,),
        mode=lax.GatherScatterMode.PROMISE_IN_BOUNDS,
    )


def _align_vreg(x, o16, base16, p1, p2, p4):
    bits = lax.bitcast_convert_type(x, jnp.int32)
    key = bits ^ (lax.shift_right_arithmetic(bits, 31) & jnp.int32(0x7FFFFFFF))
    key = (key & jnp.int32(-8)) | (7 - o16)
    key = jnp.maximum(key, _vgather(key, p1))
    key = jnp.maximum(key, _vgather(key, p2))
    key = jnp.maximum(key, _vgather(key, p4))
    d = 7 - (key & 7)
    idx = base16 + ((o16 + d) & 7)
    return _vgather(x, idx)


def kernel(input):
    N, C = input.shape
    rows_w = N // NW
    n_chunks = rows_w // CR
    vregs_row = C // LANES

    mesh = plsc.VectorSubcoreMesh(core_axis_name="c", subcore_axis_name="s")

    @functools.partial(
        pl.kernel,
        mesh=mesh,
        out_type=jax.ShapeDtypeStruct((N, C), jnp.float32),
        scratch_types=[
            pltpu.VMEM((NBUF, CR, C), jnp.float32),
            pltpu.SemaphoreType.DMA,
            pltpu.SemaphoreType.DMA,
            pltpu.SemaphoreType.DMA,
            pltpu.SemaphoreType.DMA,
            pltpu.SemaphoreType.DMA,
            pltpu.SemaphoreType.DMA,
        ],
        compiler_params=pltpu.CompilerParams(use_tc_tiling_on_sc=True),
    )
    def run(x_hbm, out_hbm, bufs, si0, si1, si2, so0, so1, so2):
        wid = lax.axis_index("s") * 2 + lax.axis_index("c")
        row0 = wid * rows_w
        iota = lax.iota(jnp.int32, LANES)
        o16 = iota & 7
        base16 = iota & jnp.int32(-8)
        p1 = iota ^ 1
        p2 = iota ^ 2
        p4 = iota ^ 4
        sem_in = (si0, si1, si2)
        sem_out = (so0, so1, so2)

        def in_slice(ci):
            return x_hbm.at[pl.ds(row0 + ci * CR, CR), :]

        def out_slice(ci):
            return out_hbm.at[pl.ds(row0 + ci * CR, CR), :]

        for b in range(2):
            pltpu.async_copy(in_slice(b), bufs.at[b], sem_in[b])

        def do_chunk(ci, b):
            buf = bufs.at[b]
            pltpu.make_async_copy(in_slice(ci), buf, sem_in[b]).wait()

            for r in range(CR):

                @plsc.parallel_loop(0, vregs_row, unroll=8)
                def body(v):
                    x = buf[r, pl.ds(v * LANES, LANES)]
                    buf[r, pl.ds(v * LANES, LANES)] = _align_vreg(
                        x, o16, base16, p1, p2, p4
                    )

            pltpu.async_copy(buf, out_slice(ci), sem_out[b])

            b2 = (b + 2) % NBUF

            @pl.when(ci + 2 < n_chunks)
            def _():
                @pl.when(ci >= 1)
                def _():
                    pltpu.make_async_copy(
                        bufs.at[b2], out_slice(ci - 1), sem_out[b2]
                    ).wait()

                pltpu.async_copy(in_slice(ci + 2), bufs.at[b2], sem_in[b2])

        def outer(t, _):
            for b in range(NBUF):
                do_chunk(t * NBUF + b, b)
            return 0

        lax.fori_loop(0, n_chunks // NBUF, outer, 0)
        for ci in range((n_chunks // NBUF) * NBUF, n_chunks):
            do_chunk(ci, ci % NBUF)

        for k in range(NBUF):
            ci = n_chunks - NBUF + k
            pltpu.make_async_copy(
                bufs.at[ci % NBUF], out_slice(ci), sem_out[ci % NBUF]
            ).wait()

    return run(input)


# speedup vs baseline: 20.7365x; 1.0266x over previous
import functools

import jax
import jax.numpy as jnp
from jax import lax
from jax.experimental import pallas as pl
from jax.experimental.pallas import tpu as pltpu
from jax.experimental.pallas import tpu_sc as plsc

LANES = 16
NW = 32
CR = 8
NBUF = 3


def _vgather(x, idx):
    return lax.gather(
        x,
        idx[:, None],
        dimension_numbers=lax.GatherDimensionNumbers(
            offset_dims=(), collapsed_slice_dims=(0,), start_index_map=(0,)
        ),
        slice_sizes=(1,),
        mode=lax.GatherScatterMode.PROMISE_IN_BOUNDS,
    )


def _align_vreg(x, o16, base16, cneg, cpos, p1, p2, p4):
    bits = lax.bitcast_convert_type(x, jnp.int32)
    km = (bits & jnp.int32(-8)) ^ jnp.where(bits < 0, cneg, cpos)
    km = jnp.minimum(km, _vgather(km, p1))
    km = jnp.minimum(km, _vgather(km, p2))
    km = jnp.minimum(km, _vgather(km, p4))
    d = km & 7
    idx = base16 | ((o16 + d) & 7)
    return _vgather(x, idx)


def kernel(input):
    N, C = input.shape
    rows_w = N // NW
    n_chunks = rows_w // CR
    vregs_row = C // LANES

    mesh = plsc.VectorSubcoreMesh(core_axis_name="c", subcore_axis_name="s")

    @functools.partial(
        pl.kernel,
        mesh=mesh,
        out_type=jax.ShapeDtypeStruct((N, C), jnp.float32),
        scratch_types=[
            pltpu.VMEM((NBUF, CR, C), jnp.float32),
            pltpu.SemaphoreType.DMA,
            pltpu.SemaphoreType.DMA,
            pltpu.SemaphoreType.DMA,
            pltpu.SemaphoreType.DMA,
            pltpu.SemaphoreType.DMA,
            pltpu.SemaphoreType.DMA,
        ],
        compiler_params=pltpu.CompilerParams(use_tc_tiling_on_sc=True),
    )
    def run(x_hbm, out_hbm, bufs, si0, si1, si2, so0, so1, so2):
        wid = lax.axis_index("s") * 2 + lax.axis_index("c")
        row0 = wid * rows_w
        iota = lax.iota(jnp.int32, LANES)
        o16 = iota & 7
        base16 = iota & jnp.int32(-8)
        p1 = iota ^ 1
        p2 = iota ^ 2
        p4 = iota ^ 4
        cneg = jnp.int32(-(2**31)) | o16
        cpos = jnp.int32(-8) ^ o16
        sem_in = (si0, si1, si2)
        sem_out = (so0, so1, so2)

        def in_slice(ci):
            return x_hbm.at[pl.ds(row0 + ci * CR, CR), :]

        def out_slice(ci):
            return out_hbm.at[pl.ds(row0 + ci * CR, CR), :]

        for b in range(2):
            pltpu.async_copy(in_slice(b), bufs.at[b], sem_in[b])

        def do_chunk(ci, b):
            buf = bufs.at[b]
            pltpu.make_async_copy(in_slice(ci), buf, sem_in[b]).wait()

            for r in range(CR):

                @plsc.parallel_loop(0, vregs_row, unroll=16)
                def body(v):
                    x = buf[r, pl.ds(v * LANES, LANES)]
                    buf[r, pl.ds(v * LANES, LANES)] = _align_vreg(
                        x, o16, base16, cneg, cpos, p1, p2, p4
                    )

            pltpu.async_copy(buf, out_slice(ci), sem_out[b])

            b2 = (b + 2) % NBUF

            @pl.when(ci + 2 < n_chunks)
            def _():
                @pl.when(ci >= 1)
                def _():
                    pltpu.make_async_copy(
                        bufs.at[b2], out_slice(ci - 1), sem_out[b2]
                    ).wait()

                pltpu.async_copy(in_slice(ci + 2), bufs.at[b2], sem_in[b2])

        def outer(t, _):
            for b in range(NBUF):
                do_chunk(t * NBUF + b, b)
            return 0

        lax.fori_loop(0, n_chunks // NBUF, outer, 0)
        for ci in range((n_chunks // NBUF) * NBUF, n_chunks):
            do_chunk(ci, ci % NBUF)

        for k in range(NBUF):
            ci = n_chunks - NBUF + k
            pltpu.make_async_copy(
                bufs.at[ci % NBUF], out_slice(ci), sem_out[ci % NBUF]
            ).wait()

    return run(input)
